# Initial kernel scaffold; baseline (speedup 1.0000x reference)
import jax
import jax.numpy as jnp
from jax.experimental import pallas as pl


def kernel(vfeat, efeat, weight, params, nbr1, nbr2):
    raise NotImplementedError("write your pallas kernel here")


# speedup vs baseline: 6.1390x; 6.1390x over previous
import functools
import math

import jax
import jax.numpy as jnp
import numpy as np
from jax import lax
from jax.experimental import pallas as pl
from jax.experimental.pallas import tpu as pltpu
from jax.experimental.pallas import tpu_sc as plsc

N_NODES = 10000
N_EDGES = 10000
D1 = 32
D2 = 32
IN_VDIM = 128
IN_EDIM = 64
OUT_VDIM = 128
OUT_EDIM = 64
DHID = 64
HEADS = 4
NUM_INDS = 4
WDIM = 16
HD = DHID // HEADS
HP = 8
KV_PAD = 256

_MH = np.zeros((DHID, HP), np.float32)
for _h in range(HEADS):
    _MH[_h * HD:(_h + 1) * HD, _h] = 1.0 / math.sqrt(DHID)
_EH = np.zeros((HP, DHID), np.float32)
for _h in range(HEADS):
    _EH[_h, _h * HD:(_h + 1) * HD] = 1.0
_E0 = np.zeros((HEADS * NUM_INDS, NUM_INDS * DHID), np.float32)
for _i in range(NUM_INDS):
    for _h in range(HEADS):
        _E0[_h * NUM_INDS + _i, _i * DHID + _h * HD:_i * DHID + (_h + 1) * HD] = 1.0
_QC_MASK = np.zeros((DHID, HEADS * NUM_INDS), np.float32)
for _c in range(HEADS * NUM_INDS):
    _QC_MASK[(_c // NUM_INDS) * HD:(_c // NUM_INDS + 1) * HD, _c] = 1.0 / math.sqrt(DHID)
_QC_COLMAP = np.array([c % NUM_INDS for c in range(HEADS * NUM_INDS)], np.int32)


def _prep_consts(params):
    c = {}
    c["WpeT"] = params["pe_v"]["W"].T
    c["bpe"] = params["pe_v"]["b"][None]
    c["Mh"] = jnp.asarray(_MH)
    c["Eh"] = jnp.asarray(_EH)
    c["E0"] = jnp.asarray(_E0)
    for name in ("isab0", "isab1"):
        p = params[name]
        m0, m1 = p["mab0"], p["mab1"]
        c[name + "_AkvqT"] = jnp.concatenate(
            [m0["k"]["W"], m0["v"]["W"], m1["q"]["W"]], axis=0).T
        c[name + "_bkvq"] = jnp.concatenate(
            [m0["k"]["b"], m0["v"]["b"], m1["q"]["b"]])[None]
        Qp0 = p["I"][0] @ m0["q"]["W"].T + m0["q"]["b"]
        c[name + "_Qp0"] = jnp.concatenate(
            [Qp0, jnp.zeros((HP - NUM_INDS, DHID), jnp.float32)], axis=0)
        c[name + "_Qc"] = Qp0.T[:, _QC_COLMAP] * _QC_MASK
        c[name + "_Wo0T"] = m0["o"]["W"].T
        c[name + "_bo0"] = m0["o"]["b"][None]
        c[name + "_Akv1T"] = jnp.concatenate(
            [m1["k"]["W"], m1["v"]["W"]], axis=0).T
        c[name + "_bkv1"] = jnp.concatenate(
            [m1["k"]["b"], m1["v"]["b"]])[None]
        c[name + "_Wo1T"] = m1["o"]["W"].T
        c[name + "_bo1"] = m1["o"]["b"][None]
    pd = params["dec_mab"]
    c["WqdT"] = pd["q"]["W"].T
    c["bqd"] = pd["q"]["b"][None]
    c["AkvdT"] = jnp.concatenate([pd["k"]["W"], pd["v"]["W"]], axis=0).T
    c["bkvd"] = jnp.concatenate([pd["k"]["b"], pd["v"]["b"]])[None]
    c["WodT"] = pd["o"]["W"].T
    c["bod"] = pd["o"]["b"][None]
    c["WdlT"] = params["dec_lin"]["W"].T
    c["bdl"] = params["dec_lin"]["b"][None]
    c["WkvT"] = jnp.concatenate(
        [params["ke_lin"]["W"], params["ve_lin"]["W"],
         jnp.zeros((KV_PAD - DHID - OUT_VDIM, DHID), jnp.float32)], axis=0).T
    c["bkv"] = jnp.concatenate(
        [params["ke_lin"]["b"], params["ve_lin"]["b"],
         jnp.zeros((KV_PAD - DHID - OUT_VDIM,), jnp.float32)])[None]
    c["WqvT"] = params["qv_lin"]["W"].T
    c["bqv"] = params["qv_lin"]["b"][None]
    return c


def _sc_gather(table, idx, chunk):
    B = idx.shape[0]
    D = table.shape[1]
    NW = 32
    b_per_w = B // NW
    n_chunks = b_per_w // chunk
    mesh = plsc.VectorSubcoreMesh(core_axis_name="c", subcore_axis_name="s")

    @functools.partial(
        pl.kernel,
        out_type=jax.ShapeDtypeStruct((B, D), jnp.float32),
        mesh=mesh,
        scratch_types=[
            pltpu.VMEM((chunk,), jnp.int32),
            pltpu.VMEM((chunk, D), jnp.float32),
            pltpu.SemaphoreType.DMA,
        ],
    )
    def gk(table_hbm, idx_hbm, out_hbm, idx_v, rows_v, sem):
        wid = lax.axis_index("s") * 2 + lax.axis_index("c")
        base0 = wid * b_per_w

        def body(t, carry):
            base = base0 + t * chunk
            pltpu.sync_copy(idx_hbm.at[pl.ds(base, chunk)], idx_v)
            pltpu.async_copy(table_hbm.at[idx_v], rows_v, sem).wait()
            pltpu.sync_copy(rows_v, out_hbm.at[pl.ds(base, chunk)])
            return carry

        lax.fori_loop(0, n_chunks, body, 0)

    return gk(table, idx)


def _isab(Xf, Bb, cr, pref):
    M = D1
    BM = Bb * M
    P = Xf @ cr[pref + "_AkvqT"] + cr[pref + "_bkvq"]
    Kp0 = P[:, :DHID]
    Vp0 = P[:, DHID:2 * DHID]
    Qp1 = P[:, 2 * DHID:]
    S0 = (Kp0 @ cr[pref + "_Qc"]).reshape(Bb, M, HEADS * NUM_INDS)
    S0 = S0 - jnp.max(S0, axis=1, keepdims=True)
    A0 = jnp.exp(S0)
    A0 = A0 / jnp.sum(A0, axis=1, keepdims=True)
    AX = A0.reshape(BM, HEADS * NUM_INDS) @ cr["E0"]
    V3 = Vp0.reshape(Bb, M, DHID)
    Qp0 = cr[pref + "_Qp0"]
    Hs = []
    for i in range(NUM_INDS):
        Ax = AX[:, i * DHID:(i + 1) * DHID].reshape(Bb, M, DHID)
        Hi = Qp0[i:i + 1, :] + jnp.sum(Ax * V3, axis=1)
        Hi = Hi + jnp.maximum(Hi @ cr[pref + "_Wo0T"] + cr[pref + "_bo0"], 0.0)
        Hs.append(Hi)
    Q3 = Qp1.reshape(Bb, M, DHID)
    KV1 = [Hs[j] @ cr[pref + "_Akv1T"] + cr[pref + "_bkv1"] for j in range(NUM_INDS)]
    Kp1 = [kv[:, :DHID] for kv in KV1]
    Vp1 = [kv[:, DHID:] for kv in KV1]
    S1 = [((Q3 * Kp1[j][:, None, :]).reshape(BM, DHID) @ cr["Mh"]).reshape(Bb, M, HP)
          for j in range(NUM_INDS)]
    mx = jnp.maximum(jnp.maximum(S1[0], S1[1]), jnp.maximum(S1[2], S1[3]))
    Ej = [jnp.exp(s - mx) for s in S1]
    den = (Ej[0] + Ej[1]) + (Ej[2] + Ej[3])
    O1 = Q3
    for j in range(NUM_INDS):
        Axf = ((Ej[j] / den).reshape(BM, HP) @ cr["Eh"]).reshape(Bb, M, DHID)
        O1 = O1 + Axf * Vp1[j][:, None, :]
    O1f = O1.reshape(BM, DHID)
    return O1f + jnp.maximum(O1f @ cr[pref + "_Wo1T"] + cr[pref + "_bo1"], 0.0)


_S1_CONST_NAMES = (
    "WpeT", "bpe", "Mh", "Eh", "E0",
    "isab0_AkvqT", "isab0_bkvq", "isab0_Qp0", "isab0_Qc", "isab0_Wo0T",
    "isab0_bo0", "isab0_Akv1T", "isab0_bkv1", "isab0_Wo1T", "isab0_bo1",
    "isab1_AkvqT", "isab1_bkvq", "isab1_Qp0", "isab1_Qc", "isab1_Wo0T",
    "isab1_bo0", "isab1_Akv1T", "isab1_bkv1", "isab1_Wo1T", "isab1_bo1",
    "WqdT", "bqd", "AkvdT", "bkvd", "WodT", "bod", "WdlT", "bdl",
    "WkvT", "bkv",
)


def _stage1_body(rows_ref, wgt_ref, ef_ref, *rest):
    const_refs = rest[:len(_S1_CONST_NAMES)]
    efn_ref, kv_ref = rest[len(_S1_CONST_NAMES):]
    cr = {n: r[...] for n, r in zip(_S1_CONST_NAMES, const_refs)}
    Bb = ef_ref.shape[0]
    M = D1
    BM = Bb * M
    X = rows_ref[...] + wgt_ref[...] @ cr["WpeT"] + cr["bpe"]
    X1 = _isab(X, Bb, cr, "isab0")
    X2 = _isab(X1, Bb, cr, "isab1")
    ef = ef_ref[...]
    Qpd = ef @ cr["WqdT"] + cr["bqd"]
    KVd = X2 @ cr["AkvdT"] + cr["bkvd"]
    Kpd = KVd[:, :DHID].reshape(Bb, M, DHID)
    Vpd = KVd[:, DHID:].reshape(Bb, M, DHID)
    Sd = ((Kpd * Qpd[:, None, :]).reshape(BM, DHID) @ cr["Mh"]).reshape(Bb, M, HP)
    Sd = Sd - jnp.max(Sd, axis=1, keepdims=True)
    Ad = jnp.exp(Sd)
    Ad = Ad / jnp.sum(Ad, axis=1, keepdims=True)
    Axd = (Ad.reshape(BM, HP) @ cr["Eh"]).reshape(Bb, M, DHID)
    Od = Qpd + jnp.sum(Axd * Vpd, axis=1)
    Od = Od + jnp.maximum(Od @ cr["WodT"] + cr["bod"], 0.0)
    efn = Od @ cr["WdlT"] + cr["bdl"]
    efn_ref[...] = efn
    kv_ref[...] = efn @ cr["WkvT"] + cr["bkv"]


def _stage1(rows, weight_f, efeat, consts, Bb):
    E = efeat.shape[0]
    grid = (E // Bb,)
    const_vals = [consts[n] for n in _S1_CONST_NAMES]

    def fullspec(a):
        return pl.BlockSpec(a.shape, lambda i: (0,) * a.ndim)

    return pl.pallas_call(
        _stage1_body,
        grid=grid,
        in_specs=[
            pl.BlockSpec((Bb * D1, IN_VDIM), lambda i: (i, 0)),
            pl.BlockSpec((Bb * D1, WDIM), lambda i: (i, 0)),
            pl.BlockSpec((Bb, IN_EDIM), lambda i: (i, 0)),
        ] + [fullspec(a) for a in const_vals],
        out_specs=[
            pl.BlockSpec((Bb, OUT_EDIM), lambda i: (i, 0)),
            pl.BlockSpec((Bb, KV_PAD), lambda i: (i, 0)),
        ],
        out_shape=[
            jax.ShapeDtypeStruct((E, OUT_EDIM), jnp.float32),
            jax.ShapeDtypeStruct((E, KV_PAD), jnp.float32),
        ],
        compiler_params=pltpu.CompilerParams(
            dimension_semantics=("arbitrary",)),
    )(rows, weight_f, efeat, *const_vals)


def _stage2_body(rows_ref, vf_ref, WqvT_ref, bqv_ref, out_ref):
    Bn = vf_ref.shape[0]
    M = D2
    q = vf_ref[...] @ WqvT_ref[...] + bqv_ref[...]
    rows = rows_ref[...]
    kn = rows[:, :DHID].reshape(Bn, M, DHID)
    s = jnp.sum(kn * q[:, None, :], axis=-1)
    s = jnp.where(s >= 0.0, s, 0.01 * s) * (1.0 / math.sqrt(DHID))
    s = s - jnp.max(s, axis=-1, keepdims=True)
    a = jnp.exp(s)
    a = a / jnp.sum(a, axis=-1, keepdims=True)
    vvn = rows[:, DHID:DHID + OUT_VDIM].reshape(Bn, M, OUT_VDIM)
    h = jnp.sum(a[:, :, None] * vvn, axis=1)
    out_ref[...] = jnp.maximum(h, 0.0)


def _stage2(rows2, vfeat, consts, Bn):
    N = vfeat.shape[0]
    grid = (N // Bn,)
    return pl.pallas_call(
        _stage2_body,
        grid=grid,
        in_specs=[
            pl.BlockSpec((Bn * D2, KV_PAD), lambda i: (i, 0)),
            pl.BlockSpec((Bn, IN_VDIM), lambda i: (i, 0)),
            pl.BlockSpec(consts["WqvT"].shape, lambda i: (0, 0)),
            pl.BlockSpec(consts["bqv"].shape, lambda i: (0, 0)),
        ],
        out_specs=pl.BlockSpec((Bn, OUT_VDIM), lambda i: (i, 0)),
        out_shape=jax.ShapeDtypeStruct((N, OUT_VDIM), jnp.float32),
        compiler_params=pltpu.CompilerParams(
            dimension_semantics=("arbitrary",)),
    )(rows2, vfeat, consts["WqvT"], consts["bqv"])


def kernel(vfeat, efeat, weight, params, nbr1, nbr2):
    consts = _prep_consts(params)
    idx1 = nbr1.astype(jnp.int32).reshape(-1)
    idx2 = nbr2.astype(jnp.int32).reshape(-1)
    rows1 = _sc_gather(vfeat, idx1, chunk=400)
    weight_f = weight.reshape(N_EDGES * D1, WDIM)
    efeat_new, kv = _stage1(rows1, weight_f, efeat, consts, Bb=200)
    rows2 = _sc_gather(kv, idx2, chunk=400)
    vfeat_new = _stage2(rows2, vfeat, consts, Bn=200)
    return (vfeat_new, efeat_new)
